# baseline (speedup 1.0000x reference)
import functools

import jax
import jax.numpy as jnp
from jax import lax
from jax.experimental import pallas as pl
from jax.experimental.pallas import tpu as pltpu
from jax.experimental.pallas import tpu_sc as plsc

N_NODES = 10000
N_HE = 2000
EMBED = 128
ENC = 256
HALF_OUT = 128
N_EDGES = 320000
HE_NNZ = 200000
SEQ = 512

NC = 2
NS = 16
NW = NC * NS
K = 128

NPAD_GC = 10112
NPAD_HE = 2048


def _epad(e):
  per = 2 * K
  return ((e // NW + per - 1) // per) * per * NW


EPAD_GC = _epad(N_EDGES)
EPAD_HE = _epad(HE_NNZ)


def _seg_sum_sc(n_table, n_acc, e_pad, nbuf):
  per_w = e_pad // NW
  n_chunks = per_w // K
  n_pairs = n_chunks // 2
  RB = nbuf * K
  rows_per_tile = n_acc // NS
  nfull, rem = divmod(rows_per_tile, RB)
  mesh = plsc.VectorSubcoreMesh(core_axis_name="c", subcore_axis_name="s")

  @functools.partial(
      pl.kernel,
      out_type=jax.ShapeDtypeStruct((NC, n_acc, EMBED), jnp.float32),
      mesh=mesh,
      scratch_types=[
          pltpu.VMEM((K,), jnp.int32),
          pltpu.VMEM((K,), jnp.int32),
          pltpu.VMEM((K,), jnp.int32),
          pltpu.VMEM((K,), jnp.int32),
          pltpu.VMEM((RB, EMBED), jnp.float32),
          pltpu.VMEM_SHARED((n_acc, EMBED), jnp.float32),
          pltpu.SemaphoreType.DMA,
          pltpu.SemaphoreType.DMA,
      ],
  )
  def k(table, gidx, sidx, out, gb0, sb0, gb1, sb1, rbuf, acc, g0, g1):
    cid = lax.axis_index("c")
    sid = lax.axis_index("s")
    wid = sid * NC + cid

    zero16 = jnp.zeros((16,), jnp.float32)

    def zrow(i, carry):
      for j in range(EMBED // 16):
        rbuf[i, pl.ds(j * 16, 16)] = zero16
      return carry

    lax.fori_loop(0, RB, zrow, 0)
    base_r = pl.multiple_of(sid * rows_per_tile, 8)
    for j in range(nfull):
      pltpu.sync_copy(rbuf, acc.at[pl.ds(base_r + j * RB, RB)])
    if rem:
      pltpu.sync_copy(rbuf.at[pl.ds(0, rem)],
                      acc.at[pl.ds(base_r + nfull * RB, rem)])
    plsc.subcore_barrier()

    ebase = wid * per_w
    rb0 = rbuf.at[pl.ds(0, K)]
    rb1 = rbuf.at[pl.ds((nbuf - 1) * K, K)]

    def coff(c):
      return pl.multiple_of(ebase + c * K, K)

    if nbuf == 2:

      def body(t, carry):
        off0 = coff(2 * t)
        off1 = coff(2 * t + 1)
        pltpu.sync_copy(gidx.at[pl.ds(off0, K)], gb0)
        pltpu.sync_copy(sidx.at[pl.ds(off0, K)], sb0)
        d0 = pltpu.async_copy(table.at[gb0], rb0, g0)
        pltpu.sync_copy(gidx.at[pl.ds(off1, K)], gb1)
        pltpu.sync_copy(sidx.at[pl.ds(off1, K)], sb1)
        d1 = pltpu.async_copy(table.at[gb1], rb1, g1)
        d0.wait()
        pltpu.sync_copy(rb0, acc.at[sb0], add=True)
        d1.wait()
        pltpu.sync_copy(rb1, acc.at[sb1], add=True)
        return carry

      lax.fori_loop(0, n_pairs, body, 0)
    else:
      def body(c, carry):
        off = coff(c)
        pltpu.sync_copy(gidx.at[pl.ds(off, K)], gb0)
        pltpu.sync_copy(sidx.at[pl.ds(off, K)], sb0)
        pltpu.async_copy(table.at[gb0], rbuf, g0).wait()
        pltpu.sync_copy(rbuf, acc.at[sb0], add=True)
        return carry

      lax.fori_loop(0, n_chunks, body, 0)
    plsc.subcore_barrier()

    for j in range(nfull):
      pltpu.sync_copy(acc.at[pl.ds(base_r + j * RB, RB)], rbuf)
      pltpu.sync_copy(rbuf, out.at[cid, pl.ds(base_r + j * RB, RB)])
    if rem:
      pltpu.sync_copy(acc.at[pl.ds(base_r + nfull * RB, rem)],
                      rbuf.at[pl.ds(0, rem)])
      pltpu.sync_copy(rbuf.at[pl.ds(0, rem)],
                      out.at[cid, pl.ds(base_r + nfull * RB, rem)])

  return k


def _counts_sc(n_acc, e_pad):
  per_t = e_pad // NS
  n_chunks = per_t // K
  n_pairs = n_chunks // 2
  rows_per_tile = n_acc // NS
  nfull, rem = divmod(rows_per_tile, K)
  mesh = plsc.VectorSubcoreMesh(core_axis_name="c", subcore_axis_name="s")

  @functools.partial(
      pl.kernel,
      out_type=jax.ShapeDtypeStruct((NC, n_acc, EMBED), jnp.float32),
      mesh=mesh,
      scratch_types=[
          pltpu.VMEM((K,), jnp.int32),
          pltpu.VMEM((K,), jnp.int32),
          pltpu.VMEM((K, EMBED), jnp.float32),
          pltpu.VMEM_SHARED((n_acc, EMBED), jnp.float32),
          pltpu.SemaphoreType.DMA,
          pltpu.SemaphoreType.DMA,
      ],
  )
  def k(idx0, idx1, out, sb0, sb1, ones, acc, s0, s1):
    cid = lax.axis_index("c")
    sid = lax.axis_index("s")

    zero16 = jnp.zeros((16,), jnp.float32)
    one16 = jnp.ones((16,), jnp.float32)

    def fill(val):
      def frow(i, carry):
        for j in range(EMBED // 16):
          ones[i, pl.ds(j * 16, 16)] = val
        return carry
      lax.fori_loop(0, K, frow, 0)

    fill(zero16)
    base_r = pl.multiple_of(sid * rows_per_tile, 8)
    for j in range(nfull):
      pltpu.sync_copy(ones, acc.at[pl.ds(base_r + j * K, K)])
    if rem:
      pltpu.sync_copy(ones.at[pl.ds(0, rem)],
                      acc.at[pl.ds(base_r + nfull * K, rem)])
    fill(one16)
    plsc.subcore_barrier()

    ebase = sid * per_t

    def body(t, carry):
      off0 = pl.multiple_of(ebase + (2 * t) * K, K)
      off1 = pl.multiple_of(ebase + (2 * t + 1) * K, K)

      @pl.when(cid == 0)
      def _():
        pltpu.sync_copy(idx0.at[pl.ds(off0, K)], sb0)
        pltpu.sync_copy(idx0.at[pl.ds(off1, K)], sb1)

      @pl.when(cid == 1)
      def _():
        pltpu.sync_copy(idx1.at[pl.ds(off0, K)], sb0)
        pltpu.sync_copy(idx1.at[pl.ds(off1, K)], sb1)

      pltpu.sync_copy(ones, acc.at[sb0], add=True)
      pltpu.sync_copy(ones, acc.at[sb1], add=True)
      return carry

    lax.fori_loop(0, n_pairs, body, 0)
    plsc.subcore_barrier()

    for j in range(nfull):
      pltpu.sync_copy(acc.at[pl.ds(base_r + j * K, K)], ones)
      pltpu.sync_copy(ones, out.at[cid, pl.ds(base_r + j * K, K)])
    if rem:
      pltpu.sync_copy(acc.at[pl.ds(base_r + nfull * K, rem)],
                      ones.at[pl.ds(0, rem)])
      pltpu.sync_copy(ones.at[pl.ds(0, rem)],
                      out.at[cid, pl.ds(base_r + nfull * K, rem)])

  return k


def _t0_body(cgc_ref, che_ref, nf_ref, hs1_ref, rso_ref, rsi_ref,
             dinv_ref, binv_ref):
  deg_out = jnp.maximum(cgc_ref[0, 0:N_NODES, 0:1], 1.0)
  rs_out = lax.rsqrt(deg_out)
  deg_in = jnp.maximum(cgc_ref[1, 0:N_NODES, 0:1], 1.0)
  rs_in = lax.rsqrt(deg_in)
  D = che_ref[0, 0:N_HE, 0:1]
  dinv = jnp.where(D > 0, 1.0 / jnp.maximum(D, 1e-12), 0.0)
  B = che_ref[1, 0:N_HE, 0:1]
  binv = jnp.where(B > 0, 1.0 / jnp.maximum(B, 1e-12), 0.0)
  hs1_ref[...] = nf_ref[...] * rs_out
  rso_ref[...] = rs_out
  rsi_ref[...] = rs_in
  dinv_ref[...] = dinv
  binv_ref[...] = binv


def _t2_body(p_ref, rsi_ref, W1_ref, b1_ref, W2_ref, hw_ref):
  s = (p_ref[0, 0:N_NODES] + p_ref[1, 0:N_NODES]) * rsi_ref[...]
  h1 = jnp.dot(s, W1_ref[...], preferred_element_type=jnp.float32) + b1_ref[...]
  hw_ref[...] = jnp.dot(h1, W2_ref[...], preferred_element_type=jnp.float32)


def _t3_body(pe_ref, binv_ref, e_ref):
  e_ref[...] = (pe_ref[0, 0:N_HE] + pe_ref[1, 0:N_HE]) * binv_ref[...]


def _hyper_combine(pn_ref, dinv_ref, b_ref, nf_ref):
  contrib = (pn_ref[0, 0:N_HE] + pn_ref[1, 0:N_HE]) * dinv_ref[...]
  full = jnp.concatenate(
      [contrib, jnp.zeros((N_NODES - N_HE, EMBED), jnp.float32)], axis=0)
  return jax.nn.relu(full + b_ref[...]) + nf_ref[...]


def _t4_body(pn_ref, dinv_ref, b_ref, nf_ref, rso_ref, hs2_ref):
  h = _hyper_combine(pn_ref, dinv_ref, b_ref, nf_ref)
  hs2_ref[...] = h * rso_ref[...]


def _t7_body(pn_ref, dinv_ref, b_ref, nf_ref, x_ref, WmW_ref, Wmb_ref,
             Wm2W_ref, Wm2b_ref, WsW_ref, Wsb_ref, WtW_ref, Wtb_ref, g_ref):
  h = _hyper_combine(pn_ref, dinv_ref, b_ref, nf_ref)
  x = x_ref[...]
  q = jax.nn.relu(
      jnp.dot(x, WmW_ref[...], preferred_element_type=jnp.float32)
      + Wmb_ref[...])
  kmat = jax.nn.relu(
      jnp.dot(h, Wm2W_ref[...], preferred_element_type=jnp.float32)
      + Wm2b_ref[...])
  C = lax.dot_general(q, kmat, (((1,), (1,)), ((), ())),
                      preferred_element_type=jnp.float32)
  m = jnp.max(C, axis=1, keepdims=True)
  E = jnp.exp(C - m)
  A = E / jnp.sum(E, axis=1, keepdims=True)
  H = jnp.dot(A, h, preferred_element_type=jnp.float32)
  cat = jnp.concatenate([x, H], axis=1)
  G1 = jax.nn.sigmoid(
      jnp.dot(cat, WsW_ref[...], preferred_element_type=jnp.float32)
      + Wsb_ref[...])
  G2 = jnp.tanh(
      jnp.dot(cat, WtW_ref[...], preferred_element_type=jnp.float32)
      + Wtb_ref[...])
  g_ref[...] = jnp.concatenate([G1, G2], axis=1)


def _tc(body, out_shape):
  return pl.pallas_call(body, out_shape=out_shape)


def _f32(*shape):
  return jax.ShapeDtypeStruct(shape, jnp.float32)


def kernel(x, node_features, edge_index, hyperedge_index, gc1_W, gc1_b,
           hc1_W, hc1_b, gc2_W, gc2_b, hc2_W, hc2_b, Wm_W, Wm_b, Wm2_W,
           Wm2_b, Ws_W, Ws_b, Wt_W, Wt_b):
  src = edge_index[0]
  dst = edge_index[1]
  node_idx = hyperedge_index[:, 0]
  edge_idx = hyperedge_index[:, 1]

  padg = jnp.zeros((EPAD_GC - N_EDGES,), jnp.int32)
  pads = jnp.full((EPAD_GC - N_EDGES,), N_NODES, jnp.int32)
  gc_g = jnp.concatenate([src, padg])
  gc_s = jnp.concatenate([dst, pads])
  padh0 = jnp.zeros((EPAD_HE - HE_NNZ,), jnp.int32)
  padht = jnp.full((EPAD_HE - HE_NNZ,), N_HE, jnp.int32)
  heA_g = jnp.concatenate([node_idx, padh0])
  heA_s = jnp.concatenate([edge_idx, padht])
  heB_g = jnp.concatenate([edge_idx, padh0])
  heB_s = jnp.concatenate([node_idx, padht])
  src_cnt = jnp.concatenate([src, pads])

  x2 = x[0]
  b1 = gc1_b.reshape(1, EMBED)
  b2 = gc2_b.reshape(1, EMBED)
  hb1 = hc1_b.reshape(1, EMBED)
  hb2 = hc2_b.reshape(1, EMBED)
  Wmb = Wm_b.reshape(1, EMBED)
  Wm2b = Wm2_b.reshape(1, EMBED)
  Wsb = Ws_b.reshape(1, HALF_OUT)
  Wtb = Wt_b.reshape(1, HALF_OUT)

  seg_gc = _seg_sum_sc(N_NODES, NPAD_GC, EPAD_GC, 1)
  seg_heA = _seg_sum_sc(N_NODES, NPAD_HE, EPAD_HE, 2)
  seg_heB = _seg_sum_sc(N_HE, NPAD_HE, EPAD_HE, 2)

  cnt_gc = _counts_sc(NPAD_GC, EPAD_GC)(src_cnt, gc_s)
  cnt_he = _counts_sc(NPAD_HE, EPAD_HE)(heB_s, heA_s)
  hs1, rs_out, rs_in, dinv, binv = _tc(
      _t0_body,
      [_f32(N_NODES, EMBED), _f32(N_NODES, 1), _f32(N_NODES, 1),
       _f32(N_HE, 1), _f32(N_HE, 1)])(cnt_gc, cnt_he, node_features)

  p1 = seg_gc(hs1, gc_g, gc_s)
  hw1 = _tc(_t2_body, _f32(N_NODES, EMBED))(p1, rs_in, gc1_W, b1, hc1_W)
  pe1 = seg_heA(hw1, heA_g, heA_s)
  e1 = _tc(_t3_body, _f32(N_HE, EMBED))(pe1, binv)
  pn1 = seg_heB(e1, heB_g, heB_s)
  hs2 = _tc(_t4_body, _f32(N_NODES, EMBED))(
      pn1, dinv, hb1, node_features, rs_out)

  p2 = seg_gc(hs2, gc_g, gc_s)
  hw2 = _tc(_t2_body, _f32(N_NODES, EMBED))(p2, rs_in, gc2_W, b2, hc2_W)
  pe2 = seg_heA(hw2, heA_g, heA_s)
  e2 = _tc(_t3_body, _f32(N_HE, EMBED))(pe2, binv)
  pn2 = seg_heB(e2, heB_g, heB_s)

  g = _tc(_t7_body, _f32(SEQ, 2 * HALF_OUT))(
      pn2, dinv, hb2, node_features, x2, Wm_W, Wmb, Wm2_W, Wm2b,
      Ws_W, Wsb, Wt_W, Wtb)
  return g.reshape(1, SEQ, 2 * HALF_OUT)


# speedup vs baseline: 1.6218x; 1.6218x over previous
import functools

import jax
import jax.numpy as jnp
from jax import lax
from jax.experimental import pallas as pl
from jax.experimental.pallas import tpu as pltpu
from jax.experimental.pallas import tpu_sc as plsc

N_NODES = 10000
N_HE = 2000
EMBED = 128
ENC = 256
HALF_OUT = 128
N_EDGES = 320000
HE_NNZ = 200000
SEQ = 512

NC = 2
NS = 16
NW = NC * NS
K = 128

NPAD_GC = 10112
NPAD_HE = 2048

def _epad(e):
  return ((e // NW + K - 1) // K) * K * NW


EPAD_GC = _epad(N_EDGES)
EPAD_HE = _epad(HE_NNZ)


def _seg_sum_sc(n_table, n_acc, e_pad):
  per_w = e_pad // NW
  n_chunks = per_w // K
  rows_per_tile = n_acc // NS
  nfull, rem = divmod(rows_per_tile, K)
  mesh = plsc.VectorSubcoreMesh(core_axis_name="c", subcore_axis_name="s")

  @functools.partial(
      pl.kernel,
      out_type=jax.ShapeDtypeStruct((NC, n_acc, EMBED), jnp.float32),
      mesh=mesh,
      scratch_types=[
          pltpu.VMEM((K,), jnp.int32),
          pltpu.VMEM((K,), jnp.int32),
          pltpu.VMEM((K, EMBED), jnp.float32),
          pltpu.VMEM_SHARED((n_acc, EMBED), jnp.float32),
          pltpu.SemaphoreType.DMA,
      ],
  )
  def k(table, gidx, sidx, out, gbuf, sbuf, rows, acc, sem):
    cid = lax.axis_index("c")
    sid = lax.axis_index("s")
    wid = sid * NC + cid

    zero16 = jnp.zeros((16,), jnp.float32)

    def zrow(i, carry):
      for j in range(EMBED // 16):
        rows[i, pl.ds(j * 16, 16)] = zero16
      return carry

    lax.fori_loop(0, K, zrow, 0)
    base_r = pl.multiple_of(sid * rows_per_tile, 8)
    for j in range(nfull):
      pltpu.sync_copy(rows, acc.at[pl.ds(base_r + j * K, K)])
    if rem:
      pltpu.sync_copy(rows.at[pl.ds(0, rem)],
                      acc.at[pl.ds(base_r + nfull * K, rem)])
    plsc.subcore_barrier()

    ebase = wid * per_w

    def body(c, carry):
      off = pl.multiple_of(ebase + c * K, 8)
      pltpu.sync_copy(gidx.at[pl.ds(off, K)], gbuf)
      pltpu.sync_copy(sidx.at[pl.ds(off, K)], sbuf)
      pltpu.async_copy(table.at[gbuf], rows, sem).wait()
      pltpu.sync_copy(rows, acc.at[sbuf], add=True)
      return carry

    lax.fori_loop(0, n_chunks, body, 0)
    plsc.subcore_barrier()

    for j in range(nfull):
      pltpu.sync_copy(acc.at[pl.ds(base_r + j * K, K)], rows)
      pltpu.sync_copy(rows, out.at[cid, pl.ds(base_r + j * K, K)])
    if rem:
      pltpu.sync_copy(acc.at[pl.ds(base_r + nfull * K, rem)],
                      rows.at[pl.ds(0, rem)])
      pltpu.sync_copy(rows.at[pl.ds(0, rem)],
                      out.at[cid, pl.ds(base_r + nfull * K, rem)])

  return k


def _counts_sc(n_acc, e_pad):
  per_w = e_pad // NS
  n_chunks = per_w // K
  rows_per_tile = n_acc // NS
  nfull, rem = divmod(rows_per_tile, K)
  mesh = plsc.VectorSubcoreMesh(core_axis_name="c", subcore_axis_name="s")

  @functools.partial(
      pl.kernel,
      out_type=jax.ShapeDtypeStruct((NC, n_acc, EMBED), jnp.float32),
      mesh=mesh,
      scratch_types=[
          pltpu.VMEM((K,), jnp.int32),
          pltpu.VMEM((K, EMBED), jnp.float32),
          pltpu.VMEM_SHARED((n_acc, EMBED), jnp.float32),
      ],
  )
  def k(idxs, out, sbuf, ones, acc):
    cid = lax.axis_index("c")
    sid = lax.axis_index("s")

    zero16 = jnp.zeros((16,), jnp.float32)
    one16 = jnp.ones((16,), jnp.float32)

    def fill(val):
      def frow(i, carry):
        for j in range(EMBED // 16):
          ones[i, pl.ds(j * 16, 16)] = val
        return carry
      lax.fori_loop(0, K, frow, 0)

    fill(zero16)
    base_r = pl.multiple_of(sid * rows_per_tile, 8)
    for j in range(nfull):
      pltpu.sync_copy(ones, acc.at[pl.ds(base_r + j * K, K)])
    if rem:
      pltpu.sync_copy(ones.at[pl.ds(0, rem)],
                      acc.at[pl.ds(base_r + nfull * K, rem)])
    fill(one16)
    plsc.subcore_barrier()

    ebase = sid * per_w

    def body(c, carry):
      off = pl.multiple_of(ebase + c * K, 8)
      pltpu.sync_copy(idxs.at[cid, pl.ds(off, K)], sbuf)
      pltpu.sync_copy(ones, acc.at[sbuf], add=True)
      return carry

    lax.fori_loop(0, n_chunks, body, 0)
    plsc.subcore_barrier()

    for j in range(nfull):
      pltpu.sync_copy(acc.at[pl.ds(base_r + j * K, K)], ones)
      pltpu.sync_copy(ones, out.at[cid, pl.ds(base_r + j * K, K)])
    if rem:
      pltpu.sync_copy(acc.at[pl.ds(base_r + nfull * K, rem)],
                      ones.at[pl.ds(0, rem)])
      pltpu.sync_copy(ones.at[pl.ds(0, rem)],
                      out.at[cid, pl.ds(base_r + nfull * K, rem)])

  return k


def _t0_body(cgc_ref, che_ref, nf_ref, hs1_ref, rso_ref, rsi_ref,
             dinv_ref, binv_ref):
  deg_out = jnp.maximum(cgc_ref[0, 0:N_NODES, 0:1], 1.0)
  rs_out = lax.rsqrt(deg_out)
  deg_in = jnp.maximum(cgc_ref[1, 0:N_NODES, 0:1], 1.0)
  rs_in = lax.rsqrt(deg_in)
  D = che_ref[0, 0:N_HE, 0:1]
  dinv = jnp.where(D > 0, 1.0 / jnp.maximum(D, 1e-12), 0.0)
  B = che_ref[1, 0:N_HE, 0:1]
  binv = jnp.where(B > 0, 1.0 / jnp.maximum(B, 1e-12), 0.0)
  hs1_ref[...] = nf_ref[...] * rs_out
  rso_ref[...] = rs_out
  rsi_ref[...] = rs_in
  dinv_ref[...] = dinv
  binv_ref[...] = binv


def _t2_body(p_ref, rsi_ref, W1_ref, b1_ref, W2_ref, hw_ref):
  s = (p_ref[0, 0:N_NODES] + p_ref[1, 0:N_NODES]) * rsi_ref[...]
  h1 = jnp.dot(s, W1_ref[...], preferred_element_type=jnp.float32) + b1_ref[...]
  hw_ref[...] = jnp.dot(h1, W2_ref[...], preferred_element_type=jnp.float32)


def _t3_body(pe_ref, binv_ref, e_ref):
  e_ref[...] = (pe_ref[0, 0:N_HE] + pe_ref[1, 0:N_HE]) * binv_ref[...]


def _hyper_combine(pn_ref, dinv_ref, b_ref, nf_ref):
  contrib = (pn_ref[0, 0:N_HE] + pn_ref[1, 0:N_HE]) * dinv_ref[...]
  full = jnp.concatenate(
      [contrib, jnp.zeros((N_NODES - N_HE, EMBED), jnp.float32)], axis=0)
  return jax.nn.relu(full + b_ref[...]) + nf_ref[...]


def _t4_body(pn_ref, dinv_ref, b_ref, nf_ref, rso_ref, hs2_ref):
  h = _hyper_combine(pn_ref, dinv_ref, b_ref, nf_ref)
  hs2_ref[...] = h * rso_ref[...]


def _t7_body(pn_ref, dinv_ref, b_ref, nf_ref, x_ref, WmW_ref, Wmb_ref,
             Wm2W_ref, Wm2b_ref, WsW_ref, Wsb_ref, WtW_ref, Wtb_ref, g_ref):
  h = _hyper_combine(pn_ref, dinv_ref, b_ref, nf_ref)
  x = x_ref[...]
  q = jax.nn.relu(
      jnp.dot(x, WmW_ref[...], preferred_element_type=jnp.float32)
      + Wmb_ref[...])
  kmat = jax.nn.relu(
      jnp.dot(h, Wm2W_ref[...], preferred_element_type=jnp.float32)
      + Wm2b_ref[...])
  C = lax.dot_general(q, kmat, (((1,), (1,)), ((), ())),
                      preferred_element_type=jnp.float32)
  m = jnp.max(C, axis=1, keepdims=True)
  E = jnp.exp(C - m)
  A = E / jnp.sum(E, axis=1, keepdims=True)
  H = jnp.dot(A, h, preferred_element_type=jnp.float32)
  cat = jnp.concatenate([x, H], axis=1)
  G1 = jax.nn.sigmoid(
      jnp.dot(cat, WsW_ref[...], preferred_element_type=jnp.float32)
      + Wsb_ref[...])
  G2 = jnp.tanh(
      jnp.dot(cat, WtW_ref[...], preferred_element_type=jnp.float32)
      + Wtb_ref[...])
  g_ref[...] = jnp.concatenate([G1, G2], axis=1)


def _tc(body, out_shape):
  return pl.pallas_call(body, out_shape=out_shape)


def _f32(*shape):
  return jax.ShapeDtypeStruct(shape, jnp.float32)


def kernel(x, node_features, edge_index, hyperedge_index, gc1_W, gc1_b,
           hc1_W, hc1_b, gc2_W, gc2_b, hc2_W, hc2_b, Wm_W, Wm_b, Wm2_W,
           Wm2_b, Ws_W, Ws_b, Wt_W, Wt_b):
  src = edge_index[0]
  dst = edge_index[1]
  node_idx = hyperedge_index[:, 0]
  edge_idx = hyperedge_index[:, 1]

  padg = jnp.zeros((EPAD_GC - N_EDGES,), jnp.int32)
  pads = jnp.full((EPAD_GC - N_EDGES,), N_NODES, jnp.int32)
  gc_g = jnp.concatenate([src, padg])
  gc_s = jnp.concatenate([dst, pads])
  padh0 = jnp.zeros((EPAD_HE - HE_NNZ,), jnp.int32)
  padht = jnp.full((EPAD_HE - HE_NNZ,), N_HE, jnp.int32)
  heA_g = jnp.concatenate([node_idx, padh0])
  heA_s = jnp.concatenate([edge_idx, padht])
  heB_g = jnp.concatenate([edge_idx, padh0])
  heB_s = jnp.concatenate([node_idx, padht])
  src_cnt = jnp.concatenate([src, pads])
  cnt_gc_idx = jnp.stack([src_cnt, gc_s])
  cnt_he_idx = jnp.stack([heB_s, heA_s])

  x2 = x[0]
  b1 = gc1_b.reshape(1, EMBED)
  b2 = gc2_b.reshape(1, EMBED)
  hb1 = hc1_b.reshape(1, EMBED)
  hb2 = hc2_b.reshape(1, EMBED)
  Wmb = Wm_b.reshape(1, EMBED)
  Wm2b = Wm2_b.reshape(1, EMBED)
  Wsb = Ws_b.reshape(1, HALF_OUT)
  Wtb = Wt_b.reshape(1, HALF_OUT)

  seg_gc = _seg_sum_sc(N_NODES, NPAD_GC, EPAD_GC)
  seg_heA = _seg_sum_sc(N_NODES, NPAD_HE, EPAD_HE)
  seg_heB = _seg_sum_sc(N_HE, NPAD_HE, EPAD_HE)

  cnt_gc = _counts_sc(NPAD_GC, EPAD_GC)(cnt_gc_idx)
  cnt_he = _counts_sc(NPAD_HE, EPAD_HE)(cnt_he_idx)
  hs1, rs_out, rs_in, dinv, binv = _tc(
      _t0_body,
      [_f32(N_NODES, EMBED), _f32(N_NODES, 1), _f32(N_NODES, 1),
       _f32(N_HE, 1), _f32(N_HE, 1)])(cnt_gc, cnt_he, node_features)

  p1 = seg_gc(hs1, gc_g, gc_s)
  hw1 = _tc(_t2_body, _f32(N_NODES, EMBED))(p1, rs_in, gc1_W, b1, hc1_W)
  pe1 = seg_heA(hw1, heA_g, heA_s)
  e1 = _tc(_t3_body, _f32(N_HE, EMBED))(pe1, binv)
  pn1 = seg_heB(e1, heB_g, heB_s)
  hs2 = _tc(_t4_body, _f32(N_NODES, EMBED))(
      pn1, dinv, hb1, node_features, rs_out)

  p2 = seg_gc(hs2, gc_g, gc_s)
  hw2 = _tc(_t2_body, _f32(N_NODES, EMBED))(p2, rs_in, gc2_W, b2, hc2_W)
  pe2 = seg_heA(hw2, heA_g, heA_s)
  e2 = _tc(_t3_body, _f32(N_HE, EMBED))(pe2, binv)
  pn2 = seg_heB(e2, heB_g, heB_s)

  g = _tc(_t7_body, _f32(SEQ, 2 * HALF_OUT))(
      pn2, dinv, hb2, node_features, x2, Wm_W, Wmb, Wm2_W, Wm2b,
      Ws_W, Wsb, Wt_W, Wtb)
  return g.reshape(1, SEQ, 2 * HALF_OUT)
